# baseline (speedup 1.0000x reference)
import jax
import jax.numpy as jnp
from jax.experimental import pallas as pl
from jax.experimental.pallas import tpu as pltpu


def _body(p_ref, tau_ref, a_ref, c_ref, out_ref):
    mu = p_ref[0]
    t0 = p_ref[5]
    t1 = t0 + p_ref[6]
    t2 = t1 + p_ref[7]

    tau = tau_ref[...]
    a = a_ref[...]
    c = c_ref[...]

    f_ble = jnp.where(a <= t0, p_ref[1],
            jnp.where(a <= t1, p_ref[2],
            jnp.where(a <= t2, p_ref[3], p_ref[4])))
    f_con = jnp.where(c == 2.0, p_ref[8],
            jnp.where(c == 3.0, p_ref[9], 0.0))
    r = tau * f_ble * f_con
    out_ref[...] = 1.0 - jnp.exp(-mu * r)


def kernel(events, params):
    n = events.shape[0]
    rows = n // 128
    ev3 = events.reshape(rows, 128, 3)
    tau = ev3[:, :, 0]
    a = ev3[:, :, 1]
    c = ev3[:, :, 2]
    R = 1024
    grid = (rows // R,)

    spec = pl.BlockSpec((R, 128), lambda i: (i, 0))
    out = pl.pallas_call(
        _body,
        grid=grid,
        in_specs=[pl.BlockSpec(memory_space=pltpu.SMEM), spec, spec, spec],
        out_specs=spec,
        out_shape=jax.ShapeDtypeStruct((rows, 128), jnp.float32),
    )(params, tau, a, c)
    return out.reshape(n)


# speedup vs baseline: 1.3459x; 1.3459x over previous
import jax
import jax.numpy as jnp
from jax.experimental import pallas as pl
from jax.experimental.pallas import tpu as pltpu


def _body(p_ref, tau_ref, a_ref, c_ref, out_ref):
    mu = p_ref[0]
    t0 = p_ref[5]
    t1 = t0 + p_ref[6]
    t2 = t1 + p_ref[7]

    tau = tau_ref[0]
    a = a_ref[0]
    c = c_ref[0]

    f_ble = jnp.where(a <= t0, p_ref[1],
            jnp.where(a <= t1, p_ref[2],
            jnp.where(a <= t2, p_ref[3], p_ref[4])))
    f_con = jnp.where(c == 2.0, p_ref[8],
            jnp.where(c == 3.0, p_ref[9], 0.0))
    r = tau * f_ble * f_con
    out_ref[...] = 1.0 - jnp.exp(-mu * r)


def kernel(events, params):
    n = events.shape[0]
    rows = n // 128
    evt = events.reshape(rows, 128, 3).transpose(2, 0, 1)
    R = 1024
    grid = (rows // R,)

    def fspec(f):
        return pl.BlockSpec((1, R, 128), lambda i, f=f: (f, i, 0))

    out = pl.pallas_call(
        _body,
        grid=grid,
        in_specs=[
            pl.BlockSpec(memory_space=pltpu.SMEM),
            fspec(0), fspec(1), fspec(2),
        ],
        out_specs=pl.BlockSpec((R, 128), lambda i: (i, 0)),
        out_shape=jax.ShapeDtypeStruct((rows, 128), jnp.float32),
    )(params, evt, evt, evt)
    return out.reshape(n)
